# Initial kernel scaffold; baseline (speedup 1.0000x reference)
import jax
import jax.numpy as jnp
from jax.experimental import pallas as pl


def kernel(pred0, pred1, pred2, dfl0, dfl1, dfl2, targets):
    raise NotImplementedError("write your pallas kernel here")


# speedup vs baseline: 1.5370x; 1.5370x over previous
import jax
import jax.numpy as jnp
from jax.experimental import pallas as pl
from jax.experimental.pallas import tpu as pltpu

NCLS = 80
RMAX = 16
BOX_GAIN, CLS_GAIN, DFL_GAIN = 7.5, 0.5, 1.5
STRIDES = (8.0, 16.0, 32.0)
EPS = 1e-07
B = 32
N = 400
SHAPES = ((64, 64), (32, 32), (16, 16))

_INTERPRET = False


def _dense_body(p0, p1, p2, o):
    i = pl.program_id(0)

    @pl.when(i == 0)
    def _():
        o[...] = jnp.zeros_like(o)

    lane = jax.lax.broadcasted_iota(jnp.int32, (1, 8), 1)
    acc = jnp.zeros((1, 8), jnp.float32)
    for s, ref in enumerate((p0, p1, p2)):
        x = ref[0]
        f = jnp.maximum(x, 0.0) + jnp.log1p(jnp.exp(-jnp.abs(x)))
        cmask = (jax.lax.broadcasted_iota(jnp.int32, x.shape, 0) >= 4)
        ssum = jnp.sum(jnp.where(cmask, f, 0.0))
        acc = acc + jnp.where(lane == s, ssum, 0.0)
    o[...] += acc


def _iou(px, py, pw, ph, tx, ty, tw, th):
    b1x1 = px - pw / 2
    b1x2 = px + pw / 2
    b1y1 = py - ph / 2
    b1y2 = py + ph / 2
    b2x1 = tx - tw / 2
    b2x2 = tx + tw / 2
    b2y1 = ty - th / 2
    b2y2 = ty + th / 2
    inter = (jnp.clip(jnp.minimum(b1x2, b2x2) - jnp.maximum(b1x1, b2x1), 0, None)
             * jnp.clip(jnp.minimum(b1y2, b2y2) - jnp.maximum(b1y1, b2y1), 0, None))
    w1, h1 = b1x2 - b1x1, b1y2 - b1y1 + EPS
    w2, h2 = b2x2 - b2x1, b2y2 - b2y1 + EPS
    union = w1 * h1 + w2 * h2 - inter + EPS
    return inter / union


def _combine_body(tt, gp0, gp1, gp2, gd0, gd1, gd2, ds, o):
    bf = tt[0:1, :]
    cf = tt[1:2, :]
    x = tt[2:3, :]
    y = tt[3:4, :]
    w_ = tt[4:5, :]
    h_ = tt[5:6, :]
    bi = bf.astype(jnp.int32)
    ci = cf.astype(jnp.int32)
    loss_box = jnp.float32(0.0)
    loss_cls = jnp.float32(0.0)
    loss_dfl = jnp.float32(0.0)
    for s, (gp, gd) in enumerate(((gp0, gd0), (gp1, gd1), (gp2, gd2))):
        H, W = SHAPES[s]
        stride = STRIDES[s]
        sw = jnp.float32(W / stride)
        sh = jnp.float32(H / stride)
        g0 = x * sw
        g1 = y * sh
        g2 = w_ * sw
        g3 = h_ * sh
        gif = jnp.floor(g0)
        gjf = jnp.floor(g1)
        gi = gif.astype(jnp.int32)
        gj = gjf.astype(jnp.int32)
        flat = bi * (H * W) + gj * W + gi
        tbx = g0 - gif
        tby = g1 - gjf
        tbw = g2
        tbh = g3
        iou = _iou(gp[0:1, :], gp[1:2, :], gp[2:3, :], gp[3:4, :],
                   tbx, tby, tbw, tbh)
        loss_box = loss_box + jnp.sum(1.0 - iou) * jnp.float32(1.0 / N)
        key = flat * NCLS + ci
        keyc = jnp.transpose(key)
        eq = (keyc == key)
        earlier = (jax.lax.broadcasted_iota(jnp.int32, (N, N), 1)
                   < jax.lax.broadcasted_iota(jnp.int32, (N, N), 0))
        dup = jnp.sum((eq & earlier).astype(jnp.int32), axis=1, keepdims=True)
        keep = jnp.transpose((dup == 0).astype(jnp.float32))
        possum = jnp.sum(gp[4:5, :] * keep)
        loss_cls = loss_cls + (ds[0, s] - possum) * jnp.float32(1.0 / (B * H * W * NCLS))
        tbxs = tbx * W
        tbys = tby * H
        tbws = tbw * W
        tbhs = tbh * H
        x1 = tbxs - tbws / 2
        y1 = tbys - tbhs / 2
        x2 = tbxs + tbws / 2
        y2 = tbys + tbhs / 2
        for j, corner in enumerate((x1, y1, x2, y2)):
            ccl = jnp.clip(corner, 0.0, float(RMAX - 1))
            tgt = jnp.clip(jnp.round(ccl), 0.0, float(RMAX - 1)).astype(jnp.int32)
            logits = gd[16 * j:16 * j + 16, :]
            m = jnp.max(logits, axis=0, keepdims=True)
            se = jnp.sum(jnp.exp(logits - m), axis=0, keepdims=True)
            lse = jnp.log(se) + m
            krow = jax.lax.broadcasted_iota(jnp.int32, (RMAX, N), 0)
            lt = jnp.sum(jnp.where(krow == tgt, logits, 0.0), axis=0, keepdims=True)
            loss_dfl = loss_dfl + jnp.sum(lse - lt)
    loss_dfl = loss_dfl * jnp.float32(1.0 / (N * 4))
    lb = loss_box * BOX_GAIN
    lc = loss_cls * CLS_GAIN
    ld = loss_dfl * DFL_GAIN
    tot = lb + lc + ld
    lane = jax.lax.broadcasted_iota(jnp.int32, (1, 4), 1)
    o[...] = jnp.where(lane == 0, tot,
                       jnp.where(lane == 1, lb, jnp.where(lane == 2, lc, ld)))


def _gather_positives(pred, dfl, targets, H, W, stride):
    bi = targets[:, 0].astype(jnp.int32)
    ci = targets[:, 1].astype(jnp.int32)
    sw = jnp.float32(W / stride)
    sh = jnp.float32(H / stride)
    gi = jnp.floor(targets[:, 2] * sw).astype(jnp.int32)
    gj = jnp.floor(targets[:, 3] * sh).astype(jnp.int32)
    cell = gj * W + gi
    predv = pred.reshape(B, 84, H * W)
    rows = predv[bi, :, cell]
    box = rows[:, 0:4]
    xcls = jnp.take_along_axis(rows, (4 + ci)[:, None], axis=1)
    gp = jnp.concatenate(
        [box, xcls, jnp.zeros((N, 3), jnp.float32)], axis=1).T
    dflv = dfl.reshape(B, 64, H * W)
    gd = dflv[bi, :, cell].T
    return gp, gd


def kernel(pred0, pred1, pred2, dfl0, dfl1, dfl2, targets):
    tt = targets.T
    gp0, gd0 = _gather_positives(pred0, dfl0, targets, 64, 64, 8.0)
    gp1, gd1 = _gather_positives(pred1, dfl1, targets, 32, 32, 16.0)
    gp2, gd2 = _gather_positives(pred2, dfl2, targets, 16, 16, 32.0)

    ds = pl.pallas_call(
        _dense_body,
        grid=(B,),
        in_specs=[
            pl.BlockSpec((1, 84, 64, 64), lambda b: (b, 0, 0, 0)),
            pl.BlockSpec((1, 84, 32, 32), lambda b: (b, 0, 0, 0)),
            pl.BlockSpec((1, 84, 16, 16), lambda b: (b, 0, 0, 0)),
        ],
        out_specs=pl.BlockSpec((1, 8), lambda b: (0, 0)),
        out_shape=jax.ShapeDtypeStruct((1, 8), jnp.float32),
        interpret=_INTERPRET,
    )(pred0, pred1, pred2)

    out = pl.pallas_call(
        _combine_body,
        out_shape=jax.ShapeDtypeStruct((1, 4), jnp.float32),
        interpret=_INTERPRET,
    )(tt, gp0, gp1, gp2, gd0, gd1, gd2, ds)
    return out.reshape(4)
